# Initial kernel scaffold; baseline (speedup 1.0000x reference)
import jax
import jax.numpy as jnp
from jax.experimental import pallas as pl


def kernel(x, params):
    raise NotImplementedError("write your pallas kernel here")


# speedup vs baseline: 8.9391x; 8.9391x over previous
import jax
import jax.numpy as jnp
from jax import lax
from jax.experimental import pallas as pl
from jax.experimental.pallas import tpu as pltpu

D_MODEL = 768
N_HEADS = 12
D_HEAD = 64
D_FFN = 1024
N_EXPERTS = 8
S = 2048
LSH_EPS = 1e-4
LN_EPS = 1e-5
F32 = jnp.float32
BF16 = jnp.bfloat16
HI = lax.Precision.HIGHEST


def _ln(x, g, b):
    m = jnp.mean(x, axis=-1, keepdims=True)
    v = jnp.mean((x - m) ** 2, axis=-1, keepdims=True)
    return (x - m) / jnp.sqrt(v + LN_EPS) * g + b


def _sort_keys(f, hw, hb):
    heads = f.reshape(1, S, N_HEADS, D_HEAD)
    hw_pad = jnp.pad(hw, ((0, 0), (0, 0), (0, 126)))
    proj = jnp.einsum('bshd,hdt->bsht', heads.astype(BF16),
                      hw_pad.astype(BF16),
                      preferred_element_type=F32)[..., :2] + hb[None, None, :, :]
    proj = proj.reshape(1, S, N_HEADS * 2)
    ang = jnp.arctan(proj[..., :N_HEADS] / (proj[..., N_HEADS:] + LSH_EPS))
    return ang[0]


def _k3_body(tc_ref, tr_ref, f_ref, x1_ref, w0_ref, w1_ref, w2_ref, cb_ref,
             y1_ref, rank_ref, xg_ref):
    nb = S // 256
    tr = tr_ref[...]
    for ib in range(nb):
        ti = tc_ref[ib * 256:(ib + 1) * 256, :]
        lt = (tr < ti)
        eq = (tr == ti)
        jidx = lax.broadcasted_iota(jnp.int32, (256, S), 1)
        iidx = lax.broadcasted_iota(jnp.int32, (256, S), 0) + ib * 256
        c = jnp.where(lt | (eq & (jidx < iidx)), 1.0, 0.0)
        rank_ref[ib * 256:(ib + 1) * 256, :] = jnp.sum(
            c, axis=1, keepdims=True).astype(jnp.int32)
    rrow = jnp.zeros((1, S), jnp.float32)
    for jb in range(nb):
        tj = tc_ref[jb * 256:(jb + 1) * 256, :]
        lt = (tj < tr)
        eq = (tj == tr)
        jidx = lax.broadcasted_iota(jnp.int32, (256, S), 0) + jb * 256
        iidx = lax.broadcasted_iota(jnp.int32, (256, S), 1)
        c = jnp.where(lt | (eq & (jidx < iidx)), 1.0, 0.0)
        rrow = rrow + jnp.sum(c, axis=0, keepdims=True)
    rank_row = rrow.astype(jnp.int32)
    f = f_ref[...]
    for sb in range(nb):
        siota = lax.broadcasted_iota(jnp.int32, (256, S), 0) + sb * 256
        pg = jnp.where(siota == rank_row, 1.0, 0.0)
        xg_ref[sb * 256:(sb + 1) * 256, :] = jnp.dot(
            pg, f, preferred_element_type=F32, precision=HI)
    xg = xg_ref[...]
    xm = jnp.concatenate([xg[S - 1:S, :], xg[:S - 1, :]], axis=0).astype(BF16)
    xp = jnp.concatenate([xg[1:S, :], xg[0:1, :]], axis=0).astype(BF16)
    y = (jnp.dot(xm, w0_ref[...].astype(BF16), preferred_element_type=F32)
         + jnp.dot(xg.astype(BF16), w1_ref[...].astype(BF16),
                   preferred_element_type=F32)
         + jnp.dot(xp, w2_ref[...].astype(BF16), preferred_element_type=F32))
    y = y + cb_ref[...]
    for ib in range(nb):
        rcol = rank_ref[ib * 256:(ib + 1) * 256, :]
        liota = lax.broadcasted_iota(jnp.int32, (256, S), 1)
        q = jnp.where(liota == rcol, 1.0, 0.0)
        y1_ref[ib * 256:(ib + 1) * 256, :] = (
            x1_ref[ib * 256:(ib + 1) * 256, :]
            + jnp.dot(q, y, preferred_element_type=F32, precision=HI))


def _k3(t, f_in, x1, w0, w1, w2, cb):
    tt = t.T
    tc = tt[:, :, None]
    tr = tt[:, None, :]
    f3 = f_in.reshape(S, N_HEADS, D_HEAD).transpose(1, 0, 2)
    x13 = x1.reshape(S, N_HEADS, D_HEAD).transpose(1, 0, 2)
    y1h = pl.pallas_call(
        _k3_body,
        grid=(N_HEADS,),
        in_specs=[
            pl.BlockSpec((None, S, 1), lambda h: (h, 0, 0)),
            pl.BlockSpec((None, 1, S), lambda h: (h, 0, 0)),
            pl.BlockSpec((None, S, D_HEAD), lambda h: (h, 0, 0)),
            pl.BlockSpec((None, S, D_HEAD), lambda h: (h, 0, 0)),
            pl.BlockSpec((None, D_HEAD, D_HEAD), lambda h: (h, 0, 0)),
            pl.BlockSpec((None, D_HEAD, D_HEAD), lambda h: (h, 0, 0)),
            pl.BlockSpec((None, D_HEAD, D_HEAD), lambda h: (h, 0, 0)),
            pl.BlockSpec((None, 1, D_HEAD), lambda h: (h, 0, 0)),
        ],
        out_specs=pl.BlockSpec((None, S, D_HEAD), lambda h: (h, 0, 0)),
        out_shape=jax.ShapeDtypeStruct((N_HEADS, S, D_HEAD), F32),
        scratch_shapes=[pltpu.VMEM((S, 1), jnp.int32),
                        pltpu.VMEM((S, D_HEAD), F32)],
    )(tc, tr, f3, x13, w0, w1, w2, cb.reshape(N_HEADS, 1, D_HEAD))
    return y1h.transpose(1, 0, 2).reshape(S, D_MODEL)


def _k5_body(gin_ref, rw_ref, rb_ref, oh_ref):
    logits = jnp.dot(gin_ref[...].astype(BF16), rw_ref[...].astype(BF16),
                     preferred_element_type=F32)[:, :N_EXPERTS] + rb_ref[...]
    m = jnp.max(logits, axis=-1, keepdims=True)
    w = 1.0 / jnp.sum(jnp.exp(logits - m), axis=-1, keepdims=True)
    idx8 = lax.broadcasted_iota(jnp.int32, (S, N_EXPERTS), 1)
    e_min = jnp.min(jnp.where(logits == m, idx8, N_EXPERTS),
                    axis=-1, keepdims=True)
    oh_ref[...] = jnp.where(idx8 == e_min, w, 0.0)


def _k5(gin, rw, rb):
    rw_pad = jnp.pad(rw, ((0, 0), (0, 120)))
    return pl.pallas_call(
        _k5_body,
        out_shape=jax.ShapeDtypeStruct((S, N_EXPERTS), F32),
    )(gin, rw_pad, rb)


def _k6_body(gin_ref, oh_ref, x2_ref, w1_ref, b1_ref, w2_ref, b2_ref, y2_ref):
    e = pl.program_id(1)

    @pl.when(e == 0)
    def _():
        y2_ref[...] = x2_ref[...]

    h = jnp.maximum(
        jnp.dot(gin_ref[...].astype(BF16), w1_ref[...].astype(BF16),
                preferred_element_type=F32) + b1_ref[...], 0.0)
    y = jnp.dot(h.astype(BF16), w2_ref[...].astype(BF16),
                preferred_element_type=F32) + b2_ref[...]
    ohc = oh_ref[...].astype(BF16).astype(F32)
    yc = y.astype(BF16).astype(F32)
    y2_ref[...] += ohc * yc


def _k6(gin, oh, x2, w1, b1, w2, b2):
    nb = S // 256
    return pl.pallas_call(
        _k6_body,
        grid=(nb, N_EXPERTS),
        in_specs=[
            pl.BlockSpec((256, D_MODEL), lambda s, e: (s, 0)),
            pl.BlockSpec((None, 256, 1), lambda s, e: (e, s, 0)),
            pl.BlockSpec((256, D_MODEL), lambda s, e: (s, 0)),
            pl.BlockSpec((None, D_MODEL, D_FFN), lambda s, e: (e, 0, 0)),
            pl.BlockSpec((None, 1, D_FFN), lambda s, e: (e, 0, 0)),
            pl.BlockSpec((None, D_FFN, D_MODEL), lambda s, e: (e, 0, 0)),
            pl.BlockSpec((None, 1, D_MODEL), lambda s, e: (e, 0, 0)),
        ],
        out_specs=pl.BlockSpec((256, D_MODEL), lambda s, e: (s, 0)),
        out_shape=jax.ShapeDtypeStruct((S, D_MODEL), F32),
    )(gin, oh.T[:, :, None], x2, w1, b1, w2, b2)


def _k7_body(a_ref, b_ref, o_ref):
    o_ref[...] = (a_ref[...] + b_ref[...]) * 0.5


def _k7(a, b):
    return pl.pallas_call(
        _k7_body, out_shape=jax.ShapeDtypeStruct((S, D_MODEL), F32))(a, b)


def kernel(x, params):
    x0 = x[0]
    x1 = jnp.repeat(x0[:, :D_MODEL // 2], 2, axis=1)
    x2 = jnp.repeat(x0[:, D_MODEL // 2:], 2, axis=1)
    p = params
    for i in range(2):
        cw = p['conv_w'][i].reshape(N_HEADS, D_HEAD, D_HEAD, 3)
        w0 = jnp.transpose(cw[..., 0], (0, 2, 1))
        w1 = jnp.transpose(cw[..., 1], (0, 2, 1))
        w2 = jnp.transpose(cw[..., 2], (0, 2, 1))
        cb = p['conv_b'][i].reshape(N_HEADS, D_HEAD)
        f_in = _ln(x2[None], p['ln1_g'][i], p['ln1_b'][i])[0]
        t = _sort_keys(f_in, p['hash_w'][i], p['hash_b'][i])
        y1 = _k3(t, f_in, x1, w0, w1, w2, cb)
        gin = _ln(y1[None], p['ln2_g'][i], p['ln2_b'][i])[0]
        oh = _k5(gin, p['router_w'][i], p['router_b'][i].reshape(1, -1))
        y2 = _k6(gin, oh, x2, p['w1'][i],
                 p['b1'][i].reshape(N_EXPERTS, 1, D_FFN),
                 p['w2'][i], p['b2'][i].reshape(N_EXPERTS, 1, D_MODEL))
        x1, x2 = y1, y2
    return _k7(x1, x2)[None]
